# baseline (speedup 1.0000x reference)
import jax
import jax.numpy as jnp
from jax.experimental import pallas as pl

_B, _N, _DIN = 64, 128, 768
_K = 1024
_L = 3
_TB = 1024


def _cdot(a, b):
    kd = a.shape[1]
    c = min(256, kd)
    acc = jnp.dot(a[:, :c], b[:c], preferred_element_type=jnp.float32)
    for k0 in range(256, kd, 256):
        acc = acc + jnp.dot(a[:, k0:k0 + 256], b[k0:k0 + 256],
                            preferred_element_type=jnp.float32)
    return acc


def _rowsum64(v):
    a = v[:, 0:8]
    for i in range(1, 8):
        a = a + v[:, 8 * i:8 * i + 8]
    a = a[:, :4] + a[:, 4:]
    a = a[:, :2] + a[:, 2:]
    return a[:, :1] + a[:, 1:]


def _tok_kernel(x_ref, cb_ref, w0_ref, b0_ref, w1_ref, b1_ref, w2_ref,
                b2_ref, w3_ref, b3_ref, sem_ref):
    h = x_ref[...]
    h = jnp.maximum(_cdot(h, w0_ref[...]) + b0_ref[...], 0.0)
    h = jnp.maximum(_cdot(h, w1_ref[...]) + b1_ref[...], 0.0)
    h = jnp.maximum(_cdot(h, w2_ref[...]) + b2_ref[...], 0.0)
    r = _cdot(h, w3_ref[...]) + b3_ref[...]

    kiota = jax.lax.broadcasted_iota(jnp.int32, (_TB, _K), 1)
    for l in range(_L):
        cb = cb_ref[l]
        c2 = _rowsum64(cb * cb).reshape(1, _K)
        s = jnp.dot(r, cb.T, preferred_element_type=jnp.float32)
        r2 = _rowsum64(r * r)
        d = r2 - 2.0 * s + c2
        dmin = jnp.min(d, axis=1, keepdims=True)
        idx = jnp.min(jnp.where(d == dmin, kiota, _K), axis=1)
        oh = (kiota == idx[:, None]).astype(jnp.float32)
        r = r - jnp.dot(oh, cb, precision=jax.lax.Precision.HIGHEST,
                        preferred_element_type=jnp.float32)
        sem_ref[:, l] = idx


def kernel(x, ids, user_ids, seq_mask, codebooks, W0, b0, W1, b1, W2, b2,
           W3, b3):
    T = _B * _N
    xf = x.reshape(T, _DIN)
    sem = pl.pallas_call(
        _tok_kernel,
        grid=(T // _TB,),
        in_specs=[
            pl.BlockSpec((_TB, _DIN), lambda i: (i, 0)),
            pl.BlockSpec((_L, _K, 64), lambda i: (0, 0, 0)),
            pl.BlockSpec((768, 512), lambda i: (0, 0)),
            pl.BlockSpec((1, 512), lambda i: (0, 0)),
            pl.BlockSpec((512, 256), lambda i: (0, 0)),
            pl.BlockSpec((1, 256), lambda i: (0, 0)),
            pl.BlockSpec((256, 128), lambda i: (0, 0)),
            pl.BlockSpec((1, 128), lambda i: (0, 0)),
            pl.BlockSpec((128, 64), lambda i: (0, 0)),
            pl.BlockSpec((1, 64), lambda i: (0, 0)),
        ],
        out_specs=pl.BlockSpec((_TB, _L), lambda i: (i, 0)),
        out_shape=jax.ShapeDtypeStruct((T, _L), jnp.int32),
    )(xf, codebooks, W0, b0.reshape(1, -1), W1, b1.reshape(1, -1), W2,
      b2.reshape(1, -1), W3, b3.reshape(1, -1))
    sem_ids = sem.reshape(_B, _N * _L)
    seq_mask_rep = jnp.repeat(seq_mask, _L, axis=1)
    return (user_ids, sem_ids, seq_mask_rep)


# speedup vs baseline: 1.1171x; 1.1171x over previous
import jax
import jax.numpy as jnp
from jax.experimental import pallas as pl

_B, _N, _DIN = 64, 128, 768
_K = 1024
_L = 3
_TB = 1024


def _cdot(a, b):
    kd = a.shape[1]
    c = min(256, kd)
    acc = jnp.dot(a[:, :c], b[:c], preferred_element_type=jnp.float32)
    for k0 in range(256, kd, 256):
        acc = acc + jnp.dot(a[:, k0:k0 + 256], b[k0:k0 + 256],
                            preferred_element_type=jnp.float32)
    return acc


def _rowsum64(v):
    a = v[:, 0:8]
    for i in range(1, 8):
        a = a + v[:, 8 * i:8 * i + 8]
    a = a[:, :4] + a[:, 4:]
    a = a[:, :2] + a[:, 2:]
    return a[:, :1] + a[:, 1:]


def _tok_kernel(x_ref, cb_ref, w0_ref, b0_ref, w1_ref, b1_ref, w2_ref,
                b2_ref, w3_ref, b3_ref, sem_ref):
    h = x_ref[...]
    h = jnp.maximum(_cdot(h, w0_ref[...]) + b0_ref[...], 0.0)
    h = jnp.maximum(_cdot(h, w1_ref[...]) + b1_ref[...], 0.0)
    h = jnp.maximum(_cdot(h, w2_ref[...]) + b2_ref[...], 0.0)
    r = _cdot(h, w3_ref[...]) + b3_ref[...]

    kiota = jax.lax.broadcasted_iota(jnp.int32, (_TB, _K), 1)
    for l in range(_L):
        cb = cb_ref[l]
        c2 = _rowsum64(cb * cb).reshape(1, _K)
        s = jnp.dot(r, cb.T, preferred_element_type=jnp.float32)
        r2 = _rowsum64(r * r)
        d = r2 - 2.0 * s + c2
        dmin = jnp.min(d, axis=1, keepdims=True)
        idx = jnp.min(jnp.where(d == dmin, kiota, _K), axis=1)
        sem_ref[:, l] = idx
        if l + 1 < _L:
            oh = (kiota == idx[:, None]).astype(jnp.float32)
            hi = cb.astype(jnp.bfloat16).astype(jnp.float32)
            rem = cb - hi
            mid = rem.astype(jnp.bfloat16).astype(jnp.float32)
            lo = rem - mid
            g = (jnp.dot(oh, hi, preferred_element_type=jnp.float32)
                 + jnp.dot(oh, mid, preferred_element_type=jnp.float32)
                 + jnp.dot(oh, lo, preferred_element_type=jnp.float32))
            r = r - g


def kernel(x, ids, user_ids, seq_mask, codebooks, W0, b0, W1, b1, W2, b2,
           W3, b3):
    T = _B * _N
    xf = x.reshape(T, _DIN)
    sem = pl.pallas_call(
        _tok_kernel,
        grid=(T // _TB,),
        in_specs=[
            pl.BlockSpec((_TB, _DIN), lambda i: (i, 0)),
            pl.BlockSpec((_L, _K, 64), lambda i: (0, 0, 0)),
            pl.BlockSpec((768, 512), lambda i: (0, 0)),
            pl.BlockSpec((1, 512), lambda i: (0, 0)),
            pl.BlockSpec((512, 256), lambda i: (0, 0)),
            pl.BlockSpec((1, 256), lambda i: (0, 0)),
            pl.BlockSpec((256, 128), lambda i: (0, 0)),
            pl.BlockSpec((1, 128), lambda i: (0, 0)),
            pl.BlockSpec((128, 64), lambda i: (0, 0)),
            pl.BlockSpec((1, 64), lambda i: (0, 0)),
        ],
        out_specs=pl.BlockSpec((_TB, _L), lambda i: (i, 0)),
        out_shape=jax.ShapeDtypeStruct((T, _L), jnp.int32),
    )(xf, codebooks, W0, b0.reshape(1, -1), W1, b1.reshape(1, -1), W2,
      b2.reshape(1, -1), W3, b3.reshape(1, -1))
    sem_ids = sem.reshape(_B, _N * _L)
    seq_mask_rep = jnp.repeat(seq_mask, _L, axis=1)
    return (user_ids, sem_ids, seq_mask_rep)


# speedup vs baseline: 1.2741x; 1.1405x over previous
import jax
import jax.numpy as jnp
from jax.experimental import pallas as pl

_B, _N, _DIN = 64, 128, 768
_K = 1024
_L = 3
_TB = 1024


def _cdot(a, b):
    kd = a.shape[1]
    c = min(256, kd)
    acc = jnp.dot(a[:, :c], b[:c], preferred_element_type=jnp.float32)
    for k0 in range(256, kd, 256):
        acc = acc + jnp.dot(a[:, k0:k0 + 256], b[k0:k0 + 256],
                            preferred_element_type=jnp.float32)
    return acc


def _rowsum64(v):
    a = v[:, 0:8]
    for i in range(1, 8):
        a = a + v[:, 8 * i:8 * i + 8]
    a = a[:, :4] + a[:, 4:]
    a = a[:, :2] + a[:, 2:]
    return a[:, :1] + a[:, 1:]


def _colsum64(v):
    a = v[0:8]
    for i in range(1, 8):
        a = a + v[8 * i:8 * i + 8]
    a = a[:4] + a[4:]
    a = a[:2] + a[2:]
    return a[:1] + a[1:]


def _tok_kernel(x_ref, cbt_ref, hi_ref, mid_ref, lo_ref, w0_ref, b0_ref,
                w1_ref, b1_ref, w2_ref, b2_ref, w3_ref, b3_ref, sem_ref):
    h = x_ref[...]
    h = jnp.maximum(_cdot(h, w0_ref[...]) + b0_ref[...], 0.0)
    h = jnp.maximum(_cdot(h, w1_ref[...]) + b1_ref[...], 0.0)
    h = jnp.maximum(_cdot(h, w2_ref[...]) + b2_ref[...], 0.0)
    r = _cdot(h, w3_ref[...]) + b3_ref[...]

    kiota = jax.lax.broadcasted_iota(jnp.int32, (_TB, _K), 1)
    for l in range(_L):
        cbt = cbt_ref[l]
        c2 = _colsum64(cbt * cbt)
        s = jnp.dot(r, cbt, preferred_element_type=jnp.float32)
        r2 = _rowsum64(r * r)
        d = r2 - 2.0 * s + c2
        dmin = jnp.min(d, axis=1, keepdims=True)
        idx = jnp.min(jnp.where(d == dmin, kiota, _K), axis=1)
        sem_ref[:, l] = idx
        if l + 1 < _L:
            oh = (kiota == idx[:, None]).astype(jnp.bfloat16)
            g = (jnp.dot(oh, hi_ref[l], preferred_element_type=jnp.float32)
                 + jnp.dot(oh, mid_ref[l], preferred_element_type=jnp.float32)
                 + jnp.dot(oh, lo_ref[l], preferred_element_type=jnp.float32))
            r = r - g


def kernel(x, ids, user_ids, seq_mask, codebooks, W0, b0, W1, b1, W2, b2,
           W3, b3):
    T = _B * _N
    xf = x.reshape(T, _DIN)
    cbt = jnp.swapaxes(codebooks, 1, 2)
    hi = codebooks.astype(jnp.bfloat16)
    rem = codebooks - hi.astype(jnp.float32)
    mid = rem.astype(jnp.bfloat16)
    lo = (rem - mid.astype(jnp.float32)).astype(jnp.bfloat16)
    sem = pl.pallas_call(
        _tok_kernel,
        grid=(T // _TB,),
        in_specs=[
            pl.BlockSpec((_TB, _DIN), lambda i: (i, 0)),
            pl.BlockSpec((_L, 64, _K), lambda i: (0, 0, 0)),
            pl.BlockSpec((_L, _K, 64), lambda i: (0, 0, 0)),
            pl.BlockSpec((_L, _K, 64), lambda i: (0, 0, 0)),
            pl.BlockSpec((_L, _K, 64), lambda i: (0, 0, 0)),
            pl.BlockSpec((768, 512), lambda i: (0, 0)),
            pl.BlockSpec((1, 512), lambda i: (0, 0)),
            pl.BlockSpec((512, 256), lambda i: (0, 0)),
            pl.BlockSpec((1, 256), lambda i: (0, 0)),
            pl.BlockSpec((256, 128), lambda i: (0, 0)),
            pl.BlockSpec((1, 128), lambda i: (0, 0)),
            pl.BlockSpec((128, 64), lambda i: (0, 0)),
            pl.BlockSpec((1, 64), lambda i: (0, 0)),
        ],
        out_specs=pl.BlockSpec((_TB, _L), lambda i: (i, 0)),
        out_shape=jax.ShapeDtypeStruct((T, _L), jnp.int32),
    )(xf, cbt, hi, mid, lo, W0, b0.reshape(1, -1), W1, b1.reshape(1, -1),
      W2, b2.reshape(1, -1), W3, b3.reshape(1, -1))
    sem_ids = sem.reshape(_B, _N * _L)
    seq_mask_rep = jnp.repeat(seq_mask, _L, axis=1)
    return (user_ids, sem_ids, seq_mask_rep)


# speedup vs baseline: 1.9901x; 1.5620x over previous
import jax
import jax.numpy as jnp
from jax.experimental import pallas as pl

_B, _N, _DIN = 64, 128, 768
_K = 1024
_L = 3
_TB = 1024
_NS = 2
_SB = _TB // _NS


def _cdot(a, b):
    kd = a.shape[1]
    c = min(256, kd)
    acc = jnp.dot(a[:, :c], b[:c], preferred_element_type=jnp.float32)
    for k0 in range(256, kd, 256):
        acc = acc + jnp.dot(a[:, k0:k0 + 256], b[k0:k0 + 256],
                            preferred_element_type=jnp.float32)
    return acc


def _rowsum64(v):
    a = v[:, 0:8]
    for i in range(1, 8):
        a = a + v[:, 8 * i:8 * i + 8]
    a = a[:, :4] + a[:, 4:]
    a = a[:, :2] + a[:, 2:]
    return a[:, :1] + a[:, 1:]


def _colsum64(v):
    a = v[0:8]
    for i in range(1, 8):
        a = a + v[8 * i:8 * i + 8]
    a = a[:4] + a[4:]
    a = a[:2] + a[2:]
    return a[:1] + a[1:]


def _tok_kernel(x_ref, cbt_ref, hi_ref, mid_ref, lo_ref, w0_ref, b0_ref,
                w1_ref, b1_ref, w2_ref, b2_ref, w3_ref, b3_ref, sem_ref):
    h = x_ref[...]
    h = jnp.maximum(_cdot(h, w0_ref[...]) + b0_ref[...], 0.0)
    h = jnp.maximum(_cdot(h, w1_ref[...]) + b1_ref[...], 0.0)
    h = jnp.maximum(_cdot(h, w2_ref[...]) + b2_ref[...], 0.0)
    z = _cdot(h, w3_ref[...]) + b3_ref[...]

    kiota = jax.lax.broadcasted_iota(jnp.int32, (_SB, _K), 1)
    rs = [z[s * _SB:(s + 1) * _SB] for s in range(_NS)]
    for l in range(_L):
        cbt = cbt_ref[l]
        c2 = _colsum64(cbt * cbt)
        for s in range(_NS):
            r = rs[s]
            sc = jnp.dot(r, cbt, preferred_element_type=jnp.float32)
            r2 = _rowsum64(r * r)
            d = r2 - 2.0 * sc + c2
            idx = jnp.argmin(d, axis=1)
            sem_ref[s * _SB:(s + 1) * _SB, l] = idx
            if l + 1 < _L:
                oh = (kiota == idx[:, None]).astype(jnp.bfloat16)
                g = (jnp.dot(oh, hi_ref[l], preferred_element_type=jnp.float32)
                     + jnp.dot(oh, mid_ref[l], preferred_element_type=jnp.float32)
                     + jnp.dot(oh, lo_ref[l], preferred_element_type=jnp.float32))
                rs[s] = r - g


def kernel(x, ids, user_ids, seq_mask, codebooks, W0, b0, W1, b1, W2, b2,
           W3, b3):
    T = _B * _N
    xf = x.reshape(T, _DIN)
    cbt = jnp.swapaxes(codebooks, 1, 2)
    hi = codebooks.astype(jnp.bfloat16)
    rem = codebooks - hi.astype(jnp.float32)
    mid = rem.astype(jnp.bfloat16)
    lo = (rem - mid.astype(jnp.float32)).astype(jnp.bfloat16)
    sem = pl.pallas_call(
        _tok_kernel,
        grid=(T // _TB,),
        in_specs=[
            pl.BlockSpec((_TB, _DIN), lambda i: (i, 0)),
            pl.BlockSpec((_L, 64, _K), lambda i: (0, 0, 0)),
            pl.BlockSpec((_L, _K, 64), lambda i: (0, 0, 0)),
            pl.BlockSpec((_L, _K, 64), lambda i: (0, 0, 0)),
            pl.BlockSpec((_L, _K, 64), lambda i: (0, 0, 0)),
            pl.BlockSpec((768, 512), lambda i: (0, 0)),
            pl.BlockSpec((1, 512), lambda i: (0, 0)),
            pl.BlockSpec((512, 256), lambda i: (0, 0)),
            pl.BlockSpec((1, 256), lambda i: (0, 0)),
            pl.BlockSpec((256, 128), lambda i: (0, 0)),
            pl.BlockSpec((1, 128), lambda i: (0, 0)),
            pl.BlockSpec((128, 64), lambda i: (0, 0)),
            pl.BlockSpec((1, 64), lambda i: (0, 0)),
        ],
        out_specs=pl.BlockSpec((_TB, _L), lambda i: (i, 0)),
        out_shape=jax.ShapeDtypeStruct((T, _L), jnp.int32),
    )(xf, cbt, hi, mid, lo, W0, b0.reshape(1, -1), W1, b1.reshape(1, -1),
      W2, b2.reshape(1, -1), W3, b3.reshape(1, -1))
    sem_ids = sem.reshape(_B, _N * _L)
    seq_mask_rep = jnp.repeat(seq_mask, _L, axis=1)
    return (user_ids, sem_ids, seq_mask_rep)
